# baseline (speedup 1.0000x reference)
import jax
import jax.numpy as jnp
from jax.experimental import pallas as pl

N_TOK = 16384
D = 10
NB = 8
BN = N_TOK // NB


def _body(xt_ref, r_ref, w1_ref, b1_ref, w2_ref, b2_ref, out_ref):
    xt = xt_ref[...]
    m = (r_ref[...] == 0).reshape(1, BN)
    wc = jnp.concatenate([w1_ref[...], w2_ref[...]], axis=0)
    yb = jax.lax.dot(wc, xt, preferred_element_type=jnp.float32)
    y1 = yb[:D] + b1_ref[...].reshape(D, 1)
    y2 = yb[D:] + b2_ref[...].reshape(D, 1)
    out_ref[...] = jnp.where(m, y1, y2)


def kernel(x, route, W1, b1, W2, b2):
    xt = x.T
    outt = pl.pallas_call(
        _body,
        grid=(NB,),
        in_specs=[
            pl.BlockSpec((D, BN), lambda i: (0, i)),
            pl.BlockSpec((BN,), lambda i: (i,)),
            pl.BlockSpec((D, D), lambda i: (0, 0)),
            pl.BlockSpec((D,), lambda i: (0,)),
            pl.BlockSpec((D, D), lambda i: (0, 0)),
            pl.BlockSpec((D,), lambda i: (0,)),
        ],
        out_specs=pl.BlockSpec((D, BN), lambda i: (0, i)),
        out_shape=jax.ShapeDtypeStruct((D, N_TOK), jnp.float32),
    )(xt, route.astype(jnp.int32), W1, b1, W2, b2)
    return outt.T


# speedup vs baseline: 2.1409x; 2.1409x over previous
import jax
import jax.numpy as jnp
from jax.experimental import pallas as pl

N_TOK = 16384
D = 10
NB = 2
BN = N_TOK // NB


def _body(xt_ref, r_ref, w1_ref, b1_ref, w2_ref, b2_ref, out_ref):
    xt = xt_ref[...]
    m = (r_ref[...] == 0).reshape(1, BN)
    wc = jnp.concatenate([w1_ref[...], w2_ref[...]], axis=0)
    yb = jax.lax.dot(wc, xt, preferred_element_type=jnp.float32)
    y1 = yb[:D] + b1_ref[...].reshape(D, 1)
    y2 = yb[D:] + b2_ref[...].reshape(D, 1)
    out_ref[...] = jnp.where(m, y1, y2)


def kernel(x, route, W1, b1, W2, b2):
    xt = x.T
    outt = pl.pallas_call(
        _body,
        grid=(NB,),
        in_specs=[
            pl.BlockSpec((D, BN), lambda i: (0, i)),
            pl.BlockSpec((BN,), lambda i: (i,)),
            pl.BlockSpec((D, D), lambda i: (0, 0)),
            pl.BlockSpec((D,), lambda i: (0,)),
            pl.BlockSpec((D, D), lambda i: (0, 0)),
            pl.BlockSpec((D,), lambda i: (0,)),
        ],
        out_specs=pl.BlockSpec((D, BN), lambda i: (0, i)),
        out_shape=jax.ShapeDtypeStruct((D, N_TOK), jnp.float32),
    )(xt, route.astype(jnp.int32), W1, b1, W2, b2)
    return outt.T
